# baseline (speedup 1.0000x reference)
import functools
import jax
import jax.numpy as jnp
from jax.experimental import pallas as pl
from jax.experimental.pallas import tpu as pltpu

_NBITS = 96
_HIDDEN = 32
_CH = 128

_NCOPY = 8
_NMSG = 4


def _body(msg_ref, even_ref, odd_ref, lat_hbm, out_hbm,
          buf_ref, copy_sem, msg_sem):
    batch = lat_hbm.shape[0]
    hw = lat_hbm.shape[2]
    cb = batch // _NCOPY
    mb = batch // _NMSG

    for c in range(_NCOPY):
        pltpu.make_async_copy(
            lat_hbm.at[pl.ds(c * cb, cb)],
            out_hbm.at[pl.ds(c * cb, cb), pl.ds(0, _CH), :],
            copy_sem.at[c],
        ).start()

    even = even_ref[...]
    odd = odd_ref[...]
    diff = odd - even
    base = jnp.sum(even, axis=0)
    msg_f = msg_ref[...].astype(jnp.float32)
    aux = jax.lax.dot_general(
        msg_f, diff, (((1,), (0,)), ((), ())),
        preferred_element_type=jnp.float32) + base[None, :]

    for m in range(_NMSG):
        slot = m % 2
        if m >= 2:
            pltpu.make_async_copy(
                buf_ref.at[slot],
                out_hbm.at[pl.ds((m - 2) * mb, mb), pl.ds(_CH, _HIDDEN), :],
                msg_sem.at[slot],
            ).wait()
        aux_c = jax.lax.slice(aux, (m * mb, 0), ((m + 1) * mb, _HIDDEN))
        buf_ref[slot] = jnp.broadcast_to(aux_c[:, :, None], (mb, _HIDDEN, hw))
        pltpu.make_async_copy(
            buf_ref.at[slot],
            out_hbm.at[pl.ds(m * mb, mb), pl.ds(_CH, _HIDDEN), :],
            msg_sem.at[slot],
        ).start()

    for m in range(_NMSG - 2, _NMSG):
        slot = m % 2
        pltpu.make_async_copy(
            buf_ref.at[slot],
            out_hbm.at[pl.ds(m * mb, mb), pl.ds(_CH, _HIDDEN), :],
            msg_sem.at[slot],
        ).wait()
    for c in range(_NCOPY):
        pltpu.make_async_copy(
            lat_hbm.at[pl.ds(c * cb, cb)],
            out_hbm.at[pl.ds(c * cb, cb), pl.ds(0, _CH), :],
            copy_sem.at[c],
        ).wait()


def kernel(latents, msg, msg_embeddings):
    batch, ch, s1, s2 = latents.shape
    hw = s1 * s2
    lat = latents.reshape(batch, ch, hw)
    even = msg_embeddings[0::2]
    odd = msg_embeddings[1::2]
    mb = batch // _NMSG

    out = pl.pallas_call(
        _body,
        in_specs=[
            pl.BlockSpec(memory_space=pltpu.VMEM),
            pl.BlockSpec(memory_space=pltpu.VMEM),
            pl.BlockSpec(memory_space=pltpu.VMEM),
            pl.BlockSpec(memory_space=pl.ANY),
        ],
        out_specs=pl.BlockSpec(memory_space=pl.ANY),
        out_shape=jax.ShapeDtypeStruct((batch, ch + _HIDDEN, hw), jnp.float32),
        scratch_shapes=[
            pltpu.VMEM((2, mb, _HIDDEN, hw), jnp.float32),
            pltpu.SemaphoreType.DMA((_NCOPY,)),
            pltpu.SemaphoreType.DMA((2,)),
        ],
    )(msg, even, odd, lat)
    return out.reshape(batch, ch + _HIDDEN, s1, s2)


# speedup vs baseline: 8.1962x; 8.1962x over previous
import functools
import jax
import jax.numpy as jnp
from jax import lax
from jax.experimental import pallas as pl
from jax.experimental.pallas import tpu as pltpu
from jax.experimental.pallas import tpu_sc as plsc

_NBITS = 96
_HIDDEN = 32
_CH = 128
_HW = 1024
_BATCH = 256

_NC = 2
_NS = 16
_NW = _NC * _NS
_RPW = _BATCH // _NW

_ROW = _CH * _HW
_OROW = (_CH + _HIDDEN) * _HW
_MSGW = _HIDDEN * _HW

_CHUNK = 32768
_CPR = _ROW // _CHUNK
_T = _RPW * _CPR


def _sc_body(lat_hbm, msg_hbm, emb_hbm, out_hbm,
             cbuf, mbuf, embv, msgv,
             in_sem, out_sem, msg_sem, stage_sem):
    wid = lax.axis_index("s") * _NC + lax.axis_index("c")
    b0 = wid * _RPW

    pltpu.make_async_copy(emb_hbm, embv, stage_sem).start()
    pltpu.make_async_copy(msg_hbm.at[pl.ds(b0, _RPW)], msgv, stage_sem).start()
    pltpu.make_async_copy(emb_hbm, embv, stage_sem).wait()
    pltpu.make_async_copy(msg_hbm.at[pl.ds(b0, _RPW)], msgv, stage_sem).wait()

    def in_copy(t, slot):
        r = t // _CPR
        c = t % _CPR
        return pltpu.make_async_copy(
            lat_hbm.at[b0 + r, pl.ds(c * _CHUNK, _CHUNK)],
            cbuf.at[slot],
            in_sem.at[slot])

    def out_copy(t, slot):
        r = t // _CPR
        c = t % _CPR
        return pltpu.make_async_copy(
            cbuf.at[slot],
            out_hbm.at[b0 + r, pl.ds(c * _CHUNK, _CHUNK)],
            out_sem.at[slot])

    def msg_copy(r):
        return pltpu.make_async_copy(
            mbuf,
            out_hbm.at[b0 + r, pl.ds(_CH * _HW, _MSGW)],
            msg_sem)

    def do_msg_row(r, first):
        def bit_chunk(c, carry):
            a0, a1 = carry
            mvec = msgv[r, pl.ds(c * 16, 16)]
            base = c * 32
            for lane in range(16):
                idx = base + 2 * lane + mvec[lane]
                a0 = a0 + embv[idx, pl.ds(0, 16)]
                a1 = a1 + embv[idx, pl.ds(16, 16)]
            return a0, a1
        zero = jnp.zeros((16,), jnp.float32)
        a0, a1 = lax.fori_loop(0, _NBITS // 16, bit_chunk, (zero, zero))
        if not first:
            msg_copy(r - 1).wait()

        for ch in range(_HIDDEN):
            src = a0 if ch < 16 else a1
            val = jnp.full((16,), src[ch % 16], jnp.float32)
            def j_body(j, _, ch=ch, val=val):
                for k in range(8):
                    mbuf[pl.ds(ch * _HW + j * 128 + k * 16, 16)] = val
                return 0
            lax.fori_loop(0, _HW // 128, j_body, 0)
        msg_copy(r).start()

    def chunk_iter(t, slot, wait_prev_out, start_next_in):
        in_copy(t, slot).wait()
        out_copy(t, slot).start()
        if wait_prev_out:
            out_copy(t - 1, 1 - slot).wait()
        if start_next_in:
            in_copy(t + 1, 1 - slot).start()

    in_copy(0, 0).start()
    for b in range(_CPR):
        chunk_iter(b, b % 2, wait_prev_out=(b > 0), start_next_in=True)
    do_msg_row(0, first=True)

    def row_body(r, _):
        t0 = r * _CPR
        for b in range(_CPR):
            chunk_iter(t0 + b, b % 2, wait_prev_out=True, start_next_in=True)
        do_msg_row(r, first=False)
        return 0
    lax.fori_loop(1, _RPW - 1, row_body, 0)

    t0 = (_RPW - 1) * _CPR
    for b in range(_CPR):
        chunk_iter(t0 + b, b % 2, wait_prev_out=True,
                   start_next_in=(b < _CPR - 1))
    do_msg_row(_RPW - 1, first=False)

    out_copy(_T - 1, (_CPR - 1) % 2).wait()
    msg_copy(_RPW - 1).wait()


def kernel(latents, msg, msg_embeddings):
    batch, ch, s1, s2 = latents.shape
    lat = latents.reshape(batch, ch * s1 * s2)

    mesh = plsc.VectorSubcoreMesh(core_axis_name="c", subcore_axis_name="s")
    k = functools.partial(
        pl.kernel,
        out_type=jax.ShapeDtypeStruct((batch, _OROW), jnp.float32),
        mesh=mesh,
        scratch_types=[
            pltpu.VMEM((2, _CHUNK), jnp.float32),
            pltpu.VMEM((_MSGW,), jnp.float32),
            pltpu.VMEM((2 * _NBITS, _HIDDEN), jnp.float32),
            pltpu.VMEM((_RPW, _NBITS), jnp.int32),
            pltpu.SemaphoreType.DMA((2,)),
            pltpu.SemaphoreType.DMA((2,)),
            pltpu.SemaphoreType.DMA,
            pltpu.SemaphoreType.DMA,
        ],
    )(_sc_body)
    out = k(lat, msg, msg_embeddings)
    return out.reshape(batch, ch + _HIDDEN, s1, s2)


# speedup vs baseline: 10.7183x; 1.3077x over previous
import functools
import jax
import jax.numpy as jnp
from jax import lax
from jax.experimental import pallas as pl
from jax.experimental.pallas import tpu as pltpu
from jax.experimental.pallas import tpu_sc as plsc

_NBITS = 96
_HIDDEN = 32
_CH = 128
_HW = 1024
_BATCH = 256

_NC = 2
_NS = 16
_NW = _NC * _NS
_RPW = _BATCH // _NW

_CCH = 32
_CPR = _CH // _CCH


def _sc_body(lat_hbm, msg_hbm, emb_hbm, out_hbm,
             cbuf, mbuf, embv, msgv,
             in_sem, out_sem, msg_sem, stage_sem):
    wid = lax.axis_index("s") * _NC + lax.axis_index("c")
    b0 = wid * _RPW

    pltpu.make_async_copy(emb_hbm, embv, stage_sem).start()
    pltpu.make_async_copy(msg_hbm.at[pl.ds(b0, _RPW)], msgv, stage_sem).start()
    pltpu.make_async_copy(emb_hbm, embv, stage_sem).wait()
    pltpu.make_async_copy(msg_hbm.at[pl.ds(b0, _RPW)], msgv, stage_sem).wait()

    def in_copy(t, slot):
        r = t // _CPR
        c = t % _CPR
        return pltpu.make_async_copy(
            lat_hbm.at[b0 + r, pl.ds(c * _CCH, _CCH), :],
            cbuf.at[slot],
            in_sem.at[slot])

    def out_copy(t, slot):
        r = t // _CPR
        c = t % _CPR
        return pltpu.make_async_copy(
            cbuf.at[slot],
            out_hbm.at[b0 + r, pl.ds(c * _CCH, _CCH), :],
            out_sem.at[slot])

    def msg_copy(r):
        return pltpu.make_async_copy(
            mbuf,
            out_hbm.at[b0 + r, pl.ds(_CH, _HIDDEN), :],
            msg_sem)

    def do_msg_row(r, first):
        def bit_chunk(c, carry):
            a0, a1 = carry
            mvec = msgv[r, pl.ds(c * 16, 16)]
            base = c * 32
            for lane in range(16):
                idx = base + 2 * lane + mvec[lane]
                a0 = a0 + embv[idx, pl.ds(0, 16)]
                a1 = a1 + embv[idx, pl.ds(16, 16)]
            return a0, a1
        zero = jnp.zeros((16,), jnp.float32)
        a0, a1 = lax.fori_loop(0, _NBITS // 16, bit_chunk, (zero, zero))
        if not first:
            msg_copy(r - 1).wait()

        for ch in range(_HIDDEN):
            src = a0 if ch < 16 else a1
            val = jnp.full((16,), src[ch % 16], jnp.float32)
            def j_body(j, _, ch=ch, val=val):
                for k in range(8):
                    mbuf[ch, pl.ds(j * 128 + k * 16, 16)] = val
                return 0
            lax.fori_loop(0, _HW // 128, j_body, 0)
        msg_copy(r).start()

    def chunk_iter(t, slot, wait_prev_out, start_next_in):
        in_copy(t, slot).wait()
        out_copy(t, slot).start()
        if wait_prev_out:
            out_copy(t - 1, 1 - slot).wait()
        if start_next_in:
            in_copy(t + 1, 1 - slot).start()

    in_copy(0, 0).start()
    for b in range(_CPR):
        chunk_iter(b, b % 2, wait_prev_out=(b > 0), start_next_in=True)
    do_msg_row(0, first=True)

    def row_body(r, _):
        t0 = r * _CPR
        for b in range(_CPR):
            chunk_iter(t0 + b, b % 2, wait_prev_out=True, start_next_in=True)
        do_msg_row(r, first=False)
        return 0
    lax.fori_loop(1, _RPW - 1, row_body, 0)

    t0 = (_RPW - 1) * _CPR
    for b in range(_CPR):
        chunk_iter(t0 + b, b % 2, wait_prev_out=True,
                   start_next_in=(b < _CPR - 1))
    do_msg_row(_RPW - 1, first=False)

    out_copy(_RPW * _CPR - 1, (_CPR - 1) % 2).wait()
    msg_copy(_RPW - 1).wait()


def kernel(latents, msg, msg_embeddings):
    batch, ch, s1, s2 = latents.shape
    lat = latents.reshape(batch, ch, s1 * s2)

    mesh = plsc.VectorSubcoreMesh(core_axis_name="c", subcore_axis_name="s")
    k = functools.partial(
        pl.kernel,
        out_type=jax.ShapeDtypeStruct((batch, ch + _HIDDEN, s1 * s2),
                                      jnp.float32),
        mesh=mesh,
        compiler_params=pltpu.CompilerParams(use_tc_tiling_on_sc=True),
        scratch_types=[
            pltpu.VMEM((2, _CCH, _HW), jnp.float32),
            pltpu.VMEM((_HIDDEN, _HW), jnp.float32),
            pltpu.VMEM((2 * _NBITS, _HIDDEN), jnp.float32),
            pltpu.VMEM((_RPW, _NBITS), jnp.int32),
            pltpu.SemaphoreType.DMA((2,)),
            pltpu.SemaphoreType.DMA((2,)),
            pltpu.SemaphoreType.DMA,
            pltpu.SemaphoreType.DMA,
        ],
    )(_sc_body)
    out = k(lat, msg, msg_embeddings)
    return out.reshape(batch, ch + _HIDDEN, s1, s2)


# speedup vs baseline: 30.3428x; 2.8309x over previous
import jax
import jax.numpy as jnp
from jax import lax
from jax.experimental import pallas as pl

_NBITS = 96
_HIDDEN = 32
_CH = 128
_HW = 1024
_PB = 16


def _body(msgT_ref, evenT_ref, oddT_ref, lat_ref, out_ref):
    evenT = evenT_ref[...]
    oddT = oddT_ref[...]
    diffT = oddT - evenT
    baseT = jnp.sum(evenT, axis=1)
    msgT = msgT_ref[...].astype(jnp.float32)
    auxT = lax.dot_general(
        diffT, msgT, (((1,), (0,)), ((), ())),
        preferred_element_type=jnp.float32) + baseT[:, None]

    x = lat_ref[...]
    for p in range(_PB):
        out_ref[pl.ds(0, _CH), p, :] = x[:, p, :].T
    b = x.shape[0]
    out_ref[pl.ds(_CH, _HIDDEN), :, :] = jnp.broadcast_to(
        auxT[:, None, :], (_HIDDEN, _PB, b))


def kernel(latents, msg, msg_embeddings):
    batch, ch, s1, s2 = latents.shape
    hw = s1 * s2
    lat = latents.transpose(0, 2, 3, 1).reshape(batch, hw, ch)
    msgT = msg.T
    evenT = msg_embeddings[0::2].T
    oddT = msg_embeddings[1::2].T

    grid = (hw // _PB,)
    outT = pl.pallas_call(
        _body,
        grid=grid,
        in_specs=[
            pl.BlockSpec((_NBITS, batch), lambda p: (0, 0)),
            pl.BlockSpec((_HIDDEN, _NBITS), lambda p: (0, 0)),
            pl.BlockSpec((_HIDDEN, _NBITS), lambda p: (0, 0)),
            pl.BlockSpec((batch, _PB, ch), lambda p: (0, p, 0)),
        ],
        out_specs=pl.BlockSpec((ch + _HIDDEN, _PB, batch), lambda p: (0, p, 0)),
        out_shape=jax.ShapeDtypeStruct((ch + _HIDDEN, hw, batch), jnp.float32),
    )(msgT, evenT, oddT, lat)
    return outT.reshape(ch + _HIDDEN, s1, s2, batch).transpose(3, 0, 1, 2)
